# Initial kernel scaffold; baseline (speedup 1.0000x reference)
import jax
import jax.numpy as jnp
from jax.experimental import pallas as pl


def kernel(x, edge_index, W1l, W1r, b1, W2l, W2r, b2):
    raise NotImplementedError("write your pallas kernel here")


# speedup vs baseline: 7.3736x; 7.3736x over previous
import functools

import jax
import jax.numpy as jnp
from jax import lax
from jax.experimental import pallas as pl
from jax.experimental.pallas import tpu as pltpu
from jax.experimental.pallas import tpu_sc as plsc

N = 10000
E = 320000
F_IN = 128
HID = 128
CLS = 32

NC = 2
NS = 16
NW = NC * NS
C = 80
CH = (E // C) // NW
CH2 = (E // C) // NS
NACC = 10240
RPT = NACC // NS
ZR = 128


def _make_sc_pass1():
    mesh = plsc.VectorSubcoreMesh(core_axis_name="c", subcore_axis_name="s")
    H = F_IN // 2
    scratch = [
        pltpu.VMEM((CH2, C), jnp.int32),
        pltpu.VMEM((CH2, C), jnp.int32),
        pltpu.VMEM((C, H), jnp.float32),
        pltpu.VMEM((ZR, H), jnp.float32),
        pltpu.VMEM_SHARED((NACC, H), jnp.float32),
    ]

    @functools.partial(
        pl.kernel, mesh=mesh,
        out_type=jax.ShapeDtypeStruct((NC, NACC, H), jnp.float32),
        compiler_params=pltpu.CompilerParams(use_tc_tiling_on_sc=False),
        scratch_types=scratch)
    def k(x2_hbm, src_hbm, dst_hbm, agg_out, srcb, dstb, rows, zbuf, acc_sh):
        c = lax.axis_index("c")
        s = lax.axis_index("s")

        @pl.loop(0, ZR)
        def _(i):
            @pl.loop(0, H, step=16)
            def _(j):
                zbuf[i, pl.ds(j, 16)] = jnp.zeros((16,), jnp.float32)

        base = s * RPT
        for kslab in range(RPT // ZR):
            pltpu.sync_copy(zbuf, acc_sh.at[pl.ds(base + kslab * ZR, ZR)])
        plsc.subcore_barrier()

        pltpu.sync_copy(src_hbm.at[s], srcb)
        pltpu.sync_copy(dst_hbm.at[s], dstb)

        @pl.loop(0, CH2)
        def _(j):
            pltpu.sync_copy(x2_hbm.at[c].at[srcb.at[j]], rows)
            pltpu.sync_copy(rows, acc_sh.at[dstb.at[j]], add=True)

        plsc.subcore_barrier()

        for kslab in range(RPT // ZR):
            r0 = base + kslab * ZR
            pltpu.sync_copy(acc_sh.at[pl.ds(r0, ZR)],
                            agg_out.at[c, pl.ds(r0, ZR)])

    return k


def _sc_segsum(width: int):
    mesh = plsc.VectorSubcoreMesh(core_axis_name="c", subcore_axis_name="s")
    scratch = [
        pltpu.VMEM((CH, C), jnp.int32),
        pltpu.VMEM((CH, C), jnp.int32),
        pltpu.VMEM((C, width), jnp.float32),
        pltpu.VMEM((ZR, width), jnp.float32),
        pltpu.VMEM_SHARED((NACC, width), jnp.float32),
    ]

    @functools.partial(
        pl.kernel, mesh=mesh,
        out_type=jax.ShapeDtypeStruct((NC, NACC, width), jnp.float32),
        compiler_params=pltpu.CompilerParams(use_tc_tiling_on_sc=False),
        scratch_types=scratch)
    def k(x_hbm, src_hbm, dst_hbm, agg_out, srcb, dstb, rows, zbuf, acc_sh):
        c = lax.axis_index("c")
        s = lax.axis_index("s")
        w = s * NC + c

        @pl.loop(0, ZR)
        def _(i):
            @pl.loop(0, width, step=16)
            def _(j):
                zbuf[i, pl.ds(j, 16)] = jnp.zeros((16,), jnp.float32)

        base = s * RPT
        for kslab in range(RPT // ZR):
            pltpu.sync_copy(zbuf, acc_sh.at[pl.ds(base + kslab * ZR, ZR)])
        plsc.subcore_barrier()

        pltpu.sync_copy(src_hbm.at[w], srcb)
        pltpu.sync_copy(dst_hbm.at[w], dstb)

        @pl.loop(0, CH)
        def _(j):
            pltpu.sync_copy(x_hbm.at[srcb.at[j]], rows)
            pltpu.sync_copy(rows, acc_sh.at[dstb.at[j]], add=True)

        plsc.subcore_barrier()

        for kslab in range(RPT // ZR):
            r0 = base + kslab * ZR
            pltpu.sync_copy(acc_sh.at[pl.ds(r0, ZR)],
                            agg_out.at[c, pl.ds(r0, ZR)])

    return k


def _make_sc_deg():
    mesh = plsc.VectorSubcoreMesh(core_axis_name="c", subcore_axis_name="s")
    scratch = [
        pltpu.VMEM((CH, C), jnp.int32),
        pltpu.VMEM((C, 16), jnp.float32),
        pltpu.VMEM((ZR, 16), jnp.float32),
        pltpu.VMEM_SHARED((NACC, 16), jnp.float32),
    ]

    @functools.partial(
        pl.kernel, mesh=mesh,
        out_type=jax.ShapeDtypeStruct((NC, NACC, 16), jnp.float32),
        compiler_params=pltpu.CompilerParams(use_tc_tiling_on_sc=False),
        scratch_types=scratch)
    def k(dst_hbm, deg_out, dstb, onesb, zdeg, deg_sh):
        c = lax.axis_index("c")
        s = lax.axis_index("s")
        w = s * NC + c

        @pl.loop(0, ZR)
        def _(i):
            zdeg[i] = jnp.zeros((16,), jnp.float32)

        @pl.loop(0, C)
        def _(i):
            onesb[i] = jnp.ones((16,), jnp.float32)

        base = s * RPT
        for kslab in range(RPT // ZR):
            pltpu.sync_copy(zdeg, deg_sh.at[pl.ds(base + kslab * ZR, ZR)])
        plsc.subcore_barrier()

        pltpu.sync_copy(dst_hbm.at[w], dstb)

        @pl.loop(0, CH)
        def _(j):
            pltpu.sync_copy(onesb, deg_sh.at[dstb.at[j]], add=True)

        plsc.subcore_barrier()

        for kslab in range(RPT // ZR):
            r0 = base + kslab * ZR
            pltpu.sync_copy(deg_sh.at[pl.ds(r0, ZR)],
                            deg_out.at[c, pl.ds(r0, ZR)])

    return k


_sc_pass1 = _make_sc_pass1()
_sc_pass2 = _sc_segsum(CLS)
_sc_deg = _make_sc_deg()


def _tc_dense1(aggp, degp, x, w1lt, w1rt, b1, w2lt, w2rt, b2):
    def body(aggp_ref, degp_ref, x_ref, w1l_ref, w1r_ref, b1_ref,
             w2l_ref, w2r_ref, b2_ref, g_ref, r_ref, dinv_ref):
        deg = degp_ref[0, :N, 0:1] + degp_ref[1, :N, 0:1]
        dinv = 1.0 / jnp.maximum(deg, 1.0)
        agg = jnp.concatenate([aggp_ref[0, :N], aggp_ref[1, :N]], axis=-1)
        mean = agg * dinv
        h = jnp.dot(mean, w1l_ref[...], preferred_element_type=jnp.float32)
        h += jnp.dot(x_ref[...], w1r_ref[...], preferred_element_type=jnp.float32)
        h = jnp.maximum(h + b1_ref[...], 0.0)
        g_ref[...] = jnp.dot(h, w2l_ref[...], preferred_element_type=jnp.float32)
        r_ref[...] = (jnp.dot(h, w2r_ref[...], preferred_element_type=jnp.float32)
                      + b2_ref[...])
        dinv_ref[...] = dinv

    return pl.pallas_call(
        body,
        out_shape=[
            jax.ShapeDtypeStruct((N, CLS), jnp.float32),
            jax.ShapeDtypeStruct((N, CLS), jnp.float32),
            jax.ShapeDtypeStruct((N, 1), jnp.float32),
        ],
    )(aggp, degp, x, w1lt, w1rt, b1, w2lt, w2rt, b2)


def _tc_finish(agg2p, dinv, r):
    def body(agg2p_ref, dinv_ref, r_ref, o_ref):
        m = (agg2p_ref[0, :N] + agg2p_ref[1, :N]) * dinv_ref[...] + r_ref[...]
        mx = jnp.max(m, axis=1, keepdims=True)
        sh = m - mx
        o_ref[...] = sh - jnp.log(jnp.sum(jnp.exp(sh), axis=1, keepdims=True))

    return pl.pallas_call(
        body,
        out_shape=jax.ShapeDtypeStruct((N, CLS), jnp.float32),
    )(agg2p, dinv, r)


def kernel(x, edge_index, W1l, W1r, b1, W2l, W2r, b2):
    src = edge_index[0].reshape(NW, CH, C)
    dst = edge_index[1].reshape(NW, CH, C)
    src16 = edge_index[0].reshape(NS, CH2, C)
    dst16 = edge_index[1].reshape(NS, CH2, C)
    x2 = jnp.moveaxis(x.reshape(N, NC, F_IN // 2), 1, 0)
    degp = _sc_deg(dst)
    aggp = _sc_pass1(x2, src16, dst16)
    g, r, dinv = _tc_dense1(aggp, degp, x, W1l.T, W1r.T, b1.reshape(1, HID),
                            W2l.T, W2r.T, b2.reshape(1, CLS))
    agg2p = _sc_pass2(g, src, dst)
    return _tc_finish(agg2p, dinv, r)


# speedup vs baseline: 12.2678x; 1.6637x over previous
import functools

import jax
import jax.numpy as jnp
from jax import lax
from jax.experimental import pallas as pl
from jax.experimental.pallas import tpu as pltpu
from jax.experimental.pallas import tpu_sc as plsc

N = 10000
E = 320000
F_IN = 128
HID = 128
CLS = 32

NC = 2
NS = 16
NW = NC * NS
C = 125
CHW = (E // C) // NW
CH2 = (E // C) // NS
NACC = 10240
RPT = NACC // NS
ZR = 128
DEPTH = 4

_SC_PARAMS = pltpu.CompilerParams(use_tc_tiling_on_sc=False)


def _zero_slab(zbuf, width):
    @pl.loop(0, ZR)
    def _(i):
        @pl.loop(0, width, step=16)
        def _(j):
            zbuf[i, pl.ds(j, 16)] = jnp.zeros((16,), jnp.float32)


def _make_sc_pass1():
    mesh = plsc.VectorSubcoreMesh(core_axis_name="c", subcore_axis_name="s")
    H = F_IN // 2
    scratch = (
        [pltpu.VMEM((CH2, C), jnp.int32),
         pltpu.VMEM((CH2, C), jnp.int32)]
        + [pltpu.VMEM((C, H), jnp.float32) for _ in range(DEPTH)]
        + [pltpu.VMEM((ZR, H), jnp.float32),
           pltpu.VMEM_SHARED((NACC, H), jnp.float32)]
        + [pltpu.SemaphoreType.DMA for _ in range(DEPTH)]
        + [pltpu.SemaphoreType.DMA]
    )

    @functools.partial(
        pl.kernel, mesh=mesh,
        out_type=jax.ShapeDtypeStruct((NC, NACC, H), jnp.float32),
        compiler_params=_SC_PARAMS,
        scratch_types=scratch)
    def k(x2_hbm, src_hbm, dst_hbm, agg_out, srcb, dstb, *rest):
        rows = rest[:DEPTH]
        zbuf = rest[DEPTH]
        acc_sh = rest[DEPTH + 1]
        gsem = rest[DEPTH + 2:2 * DEPTH + 2]
        ssem = rest[2 * DEPTH + 2]
        c = lax.axis_index("c")
        s = lax.axis_index("s")

        _zero_slab(zbuf, H)
        base = s * RPT
        for kslab in range(RPT // ZR):
            pltpu.sync_copy(zbuf, acc_sh.at[pl.ds(base + kslab * ZR, ZR)])
        plsc.subcore_barrier()

        pltpu.sync_copy(src_hbm.at[s], srcb)
        pltpu.sync_copy(dst_hbm.at[s], dstb)

        @pl.loop(0, CH2, step=DEPTH)
        def _(j):
            gh = [pltpu.async_copy(x2_hbm.at[c].at[srcb.at[j + b]],
                                   rows[b], gsem[b])
                  for b in range(DEPTH)]
            sh = []
            for b in range(DEPTH):
                gh[b].wait()
                sh.append(pltpu.async_copy(rows[b], acc_sh.at[dstb.at[j + b]],
                                           ssem, add=True))
            for b in range(DEPTH):
                sh[b].wait()

        plsc.subcore_barrier()

        for kslab in range(RPT // ZR):
            r0 = base + kslab * ZR
            pltpu.sync_copy(acc_sh.at[pl.ds(r0, ZR)],
                            agg_out.at[c, pl.ds(r0, ZR)])

    return k


def _make_sc_pass2():
    mesh = plsc.VectorSubcoreMesh(core_axis_name="c", subcore_axis_name="s")
    W = CLS
    scratch = (
        [pltpu.VMEM((CHW, C), jnp.int32),
         pltpu.VMEM((CHW, C), jnp.int32)]
        + [pltpu.VMEM((C, W), jnp.float32) for _ in range(DEPTH)]
        + [pltpu.VMEM((ZR, W), jnp.float32),
           pltpu.VMEM_SHARED((NACC, W), jnp.float32)]
        + [pltpu.SemaphoreType.DMA for _ in range(DEPTH)]
        + [pltpu.SemaphoreType.DMA]
    )

    @functools.partial(
        pl.kernel, mesh=mesh,
        out_type=jax.ShapeDtypeStruct((NC, NACC, W), jnp.float32),
        compiler_params=_SC_PARAMS,
        scratch_types=scratch)
    def k(g_hbm, src_hbm, dst_hbm, agg_out, srcb, dstb, *rest):
        rows = rest[:DEPTH]
        zbuf = rest[DEPTH]
        acc_sh = rest[DEPTH + 1]
        gsem = rest[DEPTH + 2:2 * DEPTH + 2]
        ssem = rest[2 * DEPTH + 2]
        c = lax.axis_index("c")
        s = lax.axis_index("s")
        w = s * NC + c

        _zero_slab(zbuf, W)
        base = s * RPT
        for kslab in range(RPT // ZR):
            pltpu.sync_copy(zbuf, acc_sh.at[pl.ds(base + kslab * ZR, ZR)])
        plsc.subcore_barrier()

        pltpu.sync_copy(src_hbm.at[w], srcb)
        pltpu.sync_copy(dst_hbm.at[w], dstb)

        @pl.loop(0, CHW, step=DEPTH)
        def _(j):
            gh = [pltpu.async_copy(g_hbm.at[srcb.at[j + b]], rows[b], gsem[b])
                  for b in range(DEPTH)]
            sh = []
            for b in range(DEPTH):
                gh[b].wait()
                sh.append(pltpu.async_copy(rows[b], acc_sh.at[dstb.at[j + b]],
                                           ssem, add=True))
            for b in range(DEPTH):
                sh[b].wait()

        plsc.subcore_barrier()

        for kslab in range(RPT // ZR):
            r0 = base + kslab * ZR
            pltpu.sync_copy(acc_sh.at[pl.ds(r0, ZR)],
                            agg_out.at[c, pl.ds(r0, ZR)])

    return k


def _make_sc_deg():
    mesh = plsc.VectorSubcoreMesh(core_axis_name="c", subcore_axis_name="s")
    scratch = [
        pltpu.VMEM((CHW, C), jnp.int32),
        pltpu.VMEM((C, 16), jnp.float32),
        pltpu.VMEM((ZR, 16), jnp.float32),
        pltpu.VMEM_SHARED((NACC, 16), jnp.float32),
        pltpu.SemaphoreType.DMA,
    ]

    @functools.partial(
        pl.kernel, mesh=mesh,
        out_type=jax.ShapeDtypeStruct((NC, NACC, 16), jnp.float32),
        compiler_params=_SC_PARAMS,
        scratch_types=scratch)
    def k(dst_hbm, deg_out, dstb, onesb, zdeg, deg_sh, ssem):
        c = lax.axis_index("c")
        s = lax.axis_index("s")
        w = s * NC + c

        @pl.loop(0, ZR)
        def _(i):
            zdeg[i] = jnp.zeros((16,), jnp.float32)

        @pl.loop(0, C)
        def _(i):
            onesb[i] = jnp.ones((16,), jnp.float32)

        base = s * RPT
        for kslab in range(RPT // ZR):
            pltpu.sync_copy(zdeg, deg_sh.at[pl.ds(base + kslab * ZR, ZR)])
        plsc.subcore_barrier()

        pltpu.sync_copy(dst_hbm.at[w], dstb)

        @pl.loop(0, CHW, step=DEPTH)
        def _(j):
            sh = [pltpu.async_copy(onesb, deg_sh.at[dstb.at[j + b]],
                                   ssem, add=True)
                  for b in range(DEPTH)]
            for b in range(DEPTH):
                sh[b].wait()

        plsc.subcore_barrier()

        for kslab in range(RPT // ZR):
            r0 = base + kslab * ZR
            pltpu.sync_copy(deg_sh.at[pl.ds(r0, ZR)],
                            deg_out.at[c, pl.ds(r0, ZR)])

    return k


_sc_pass1 = _make_sc_pass1()
_sc_pass2 = _make_sc_pass2()
_sc_deg = _make_sc_deg()


def _tc_dense1a(x, w1rt, b1):
    def body(x_ref, w_ref, b_ref, o_ref):
        o_ref[...] = jnp.dot(x_ref[...], w_ref[...],
                             preferred_element_type=jnp.float32) + b_ref[...]

    return pl.pallas_call(
        body, out_shape=jax.ShapeDtypeStruct((N, HID), jnp.float32),
    )(x, w1rt, b1)


def _tc_dense1b(aggp, degp, xr, w1lt, w2lt, w2rt, b2):
    def body(aggp_ref, degp_ref, xr_ref, w1l_ref, w2l_ref, w2r_ref, b2_ref,
             g_ref, r_ref, dinv_ref):
        deg = degp_ref[0, :N, 0:1] + degp_ref[1, :N, 0:1]
        dinv = 1.0 / jnp.maximum(deg, 1.0)
        agg = jnp.concatenate([aggp_ref[0, :N], aggp_ref[1, :N]], axis=-1)
        mean = agg * dinv
        h = jnp.dot(mean, w1l_ref[...], preferred_element_type=jnp.float32)
        h = jnp.maximum(h + xr_ref[...], 0.0)
        g_ref[...] = jnp.dot(h, w2l_ref[...], preferred_element_type=jnp.float32)
        r_ref[...] = (jnp.dot(h, w2r_ref[...], preferred_element_type=jnp.float32)
                      + b2_ref[...])
        dinv_ref[...] = dinv

    return pl.pallas_call(
        body,
        out_shape=[
            jax.ShapeDtypeStruct((N, CLS), jnp.float32),
            jax.ShapeDtypeStruct((N, CLS), jnp.float32),
            jax.ShapeDtypeStruct((N, 1), jnp.float32),
        ],
    )(aggp, degp, xr, w1lt, w2lt, w2rt, b2)


def _tc_finish(agg2p, dinv, r):
    def body(agg2p_ref, dinv_ref, r_ref, o_ref):
        m = (agg2p_ref[0, :N] + agg2p_ref[1, :N]) * dinv_ref[...] + r_ref[...]
        mx = jnp.max(m, axis=1, keepdims=True)
        sh = m - mx
        o_ref[...] = sh - jnp.log(jnp.sum(jnp.exp(sh), axis=1, keepdims=True))

    return pl.pallas_call(
        body,
        out_shape=jax.ShapeDtypeStruct((N, CLS), jnp.float32),
    )(agg2p, dinv, r)


def kernel(x, edge_index, W1l, W1r, b1, W2l, W2r, b2):
    src32 = edge_index[0].reshape(NW, CHW, C)
    dst32 = edge_index[1].reshape(NW, CHW, C)
    src16 = edge_index[0].reshape(NS, CH2, C)
    dst16 = edge_index[1].reshape(NS, CH2, C)
    x2 = jnp.moveaxis(x.reshape(N, NC, F_IN // 2), 1, 0)
    degp = _sc_deg(dst32)
    aggp = _sc_pass1(x2, src16, dst16)
    xr = _tc_dense1a(x, W1r.T, b1.reshape(1, HID))
    g, r, dinv = _tc_dense1b(aggp, degp, xr, W1l.T, W2l.T, W2r.T,
                             b2.reshape(1, CLS))
    agg2p = _sc_pass2(g, src32, dst32)
    return _tc_finish(agg2p, dinv, r)


# speedup vs baseline: 12.7382x; 1.0383x over previous
import functools

import jax
import jax.numpy as jnp
from jax import lax
from jax.experimental import pallas as pl
from jax.experimental.pallas import tpu as pltpu
from jax.experimental.pallas import tpu_sc as plsc

N = 10000
E = 320000
F_IN = 128
HID = 128
CLS = 32

NC = 2
NS = 16
NW = NC * NS
C = 125
CHW = (E // C) // NW
CH2 = (E // C) // NS
NACC = 10240
RPT = NACC // NS
ZR = 128
DEPTH1 = 5
DEPTH2 = 8
HCH = CH2 // 2
ZB = 32

_SC_PARAMS = pltpu.CompilerParams(use_tc_tiling_on_sc=False)


def _zero_slab(zbuf, width):
    @pl.loop(0, ZB)
    def _(i):
        @pl.loop(0, width, step=16)
        def _(j):
            zbuf[i, pl.ds(j, 16)] = jnp.zeros((16,), jnp.float32)


def _make_sc_pass1():
    mesh = plsc.VectorSubcoreMesh(core_axis_name="c", subcore_axis_name="s")
    H = F_IN // 2
    scratch = (
        [pltpu.VMEM((HCH, C), jnp.int32),
         pltpu.VMEM((HCH, C), jnp.int32)]
        + [pltpu.VMEM((C, H), jnp.float32) for _ in range(DEPTH1)]
        + [pltpu.VMEM((ZB, H), jnp.float32),
           pltpu.VMEM((C, 16), jnp.float32),
           pltpu.VMEM((ZB, 16), jnp.float32),
           pltpu.VMEM_SHARED((NACC, H), jnp.float32),
           pltpu.VMEM_SHARED((NACC, 16), jnp.float32)]
        + [pltpu.SemaphoreType.DMA for _ in range(DEPTH1)]
        + [pltpu.SemaphoreType.DMA]
    )

    @functools.partial(
        pl.kernel, mesh=mesh,
        out_type=[jax.ShapeDtypeStruct((NC, NACC, H), jnp.float32),
                  jax.ShapeDtypeStruct((NC, NACC, 16), jnp.float32)],
        compiler_params=_SC_PARAMS,
        scratch_types=scratch)
    def k(xf_hbm, src_hbm, dst_hbm, agg_out, deg_out,
          srcb, dstb, *rest):
        rows = rest[:DEPTH1]
        zbuf, onesb, zdeg, acc_sh, deg_sh = rest[DEPTH1:DEPTH1 + 5]
        gsem = rest[DEPTH1 + 5:2 * DEPTH1 + 5]
        ssem = rest[2 * DEPTH1 + 5]
        c = lax.axis_index("c")
        s = lax.axis_index("s")

        _zero_slab(zbuf, H)

        @pl.loop(0, ZB)
        def _(i):
            zdeg[i] = jnp.zeros((16,), jnp.float32)

        @pl.loop(0, C)
        def _(i):
            onesb[i] = jnp.ones((16,), jnp.float32)

        base = s * RPT
        for kslab in range(RPT // ZB):
            r0 = base + kslab * ZB
            pltpu.sync_copy(zbuf, acc_sh.at[pl.ds(r0, ZB)])
            pltpu.sync_copy(zdeg, deg_sh.at[pl.ds(r0, ZB)])
        plsc.subcore_barrier()

        for half_i in range(2):
            pltpu.sync_copy(src_hbm.at[c, s, pl.ds(half_i * HCH, HCH)], srcb)
            pltpu.sync_copy(dst_hbm.at[s, pl.ds(half_i * HCH, HCH)], dstb)

            @pl.loop(0, HCH, step=DEPTH1)
            def _(j):
                gh = [pltpu.async_copy(xf_hbm.at[srcb.at[j + b]],
                                       rows[b], gsem[b])
                      for b in range(DEPTH1)]
                sh = []
                for b in range(DEPTH1):
                    gh[b].wait()
                    sh.append(pltpu.async_copy(rows[b],
                                               acc_sh.at[dstb.at[j + b]],
                                               ssem, add=True))

                    @pl.when(c == half_i)
                    def _():
                        pltpu.sync_copy(onesb, deg_sh.at[dstb.at[j + b]],
                                        add=True)

                for b in range(DEPTH1):
                    sh[b].wait()

        plsc.subcore_barrier()

        for kslab in range(RPT // ZR):
            r0 = base + kslab * ZR
            pltpu.sync_copy(acc_sh.at[pl.ds(r0, ZR)],
                            agg_out.at[c, pl.ds(r0, ZR)])
            pltpu.sync_copy(deg_sh.at[pl.ds(r0, ZR)],
                            deg_out.at[c, pl.ds(r0, ZR)])

    return k


def _make_sc_pass2():
    mesh = plsc.VectorSubcoreMesh(core_axis_name="c", subcore_axis_name="s")
    W = CLS // 2
    scratch = (
        [pltpu.VMEM((CH2, C), jnp.int32),
         pltpu.VMEM((CH2, C), jnp.int32)]
        + [pltpu.VMEM((C, W), jnp.float32) for _ in range(DEPTH2)]
        + [pltpu.VMEM((ZB, W), jnp.float32),
           pltpu.VMEM_SHARED((NACC, W), jnp.float32)]
        + [pltpu.SemaphoreType.DMA for _ in range(DEPTH2)]
        + [pltpu.SemaphoreType.DMA]
    )

    @functools.partial(
        pl.kernel, mesh=mesh,
        out_type=jax.ShapeDtypeStruct((NC, NACC, W), jnp.float32),
        compiler_params=_SC_PARAMS,
        scratch_types=scratch)
    def k(gf_hbm, src_hbm, dst_hbm, agg_out, srcb, dstb, *rest):
        rows = rest[:DEPTH2]
        zbuf = rest[DEPTH2]
        acc_sh = rest[DEPTH2 + 1]
        gsem = rest[DEPTH2 + 2:2 * DEPTH2 + 2]
        ssem = rest[2 * DEPTH2 + 2]
        c = lax.axis_index("c")
        s = lax.axis_index("s")

        _zero_slab(zbuf, W)
        base = s * RPT
        for kslab in range(RPT // ZB):
            pltpu.sync_copy(zbuf, acc_sh.at[pl.ds(base + kslab * ZB, ZB)])
        plsc.subcore_barrier()

        pltpu.sync_copy(src_hbm.at[c, s], srcb)
        pltpu.sync_copy(dst_hbm.at[s], dstb)

        @pl.loop(0, CH2, step=DEPTH2)
        def _(j):
            gh = [pltpu.async_copy(gf_hbm.at[srcb.at[j + b]],
                                   rows[b], gsem[b])
                  for b in range(DEPTH2)]
            sh = []
            for b in range(DEPTH2):
                gh[b].wait()
                sh.append(pltpu.async_copy(rows[b], acc_sh.at[dstb.at[j + b]],
                                           ssem, add=True))
            for b in range(DEPTH2):
                sh[b].wait()

        plsc.subcore_barrier()

        for kslab in range(RPT // ZR):
            r0 = base + kslab * ZR
            pltpu.sync_copy(acc_sh.at[pl.ds(r0, ZR)],
                            agg_out.at[c, pl.ds(r0, ZR)])

    return k


_sc_pass1 = _make_sc_pass1()
_sc_pass2 = _make_sc_pass2()


def _tc_prep(srcr):
    def body(s_ref, o_ref):
        s2 = s_ref[...] * 2
        o_ref[0] = s2
        o_ref[1] = s2 + 1

    return pl.pallas_call(
        body,
        out_shape=jax.ShapeDtypeStruct((NC,) + srcr.shape, jnp.int32),
    )(srcr)


def _tc_dense1a(x, w1rt, b1):
    def body(x_ref, w_ref, b_ref, o_ref):
        o_ref[...] = jnp.dot(x_ref[...], w_ref[...],
                             preferred_element_type=jnp.float32) + b_ref[...]

    return pl.pallas_call(
        body, out_shape=jax.ShapeDtypeStruct((N, HID), jnp.float32),
    )(x, w1rt, b1)


def _tc_dense1b(aggp, degp, xr, w1lt, w2lt, w2rt, b2):
    def body(aggp_ref, degp_ref, xr_ref, w1l_ref, w2l_ref, w2r_ref, b2_ref,
             g_ref, r_ref, dinv_ref):
        deg = degp_ref[0, :N, 0:1] + degp_ref[1, :N, 0:1]
        dinv = 1.0 / jnp.maximum(deg, 1.0)
        agg = jnp.concatenate([aggp_ref[0, :N], aggp_ref[1, :N]], axis=-1)
        mean = agg * dinv
        h = jnp.dot(mean, w1l_ref[...], preferred_element_type=jnp.float32)
        h = jnp.maximum(h + xr_ref[...], 0.0)
        g_ref[...] = jnp.dot(h, w2l_ref[...], preferred_element_type=jnp.float32)
        r_ref[...] = (jnp.dot(h, w2r_ref[...], preferred_element_type=jnp.float32)
                      + b2_ref[...])
        dinv_ref[...] = dinv

    return pl.pallas_call(
        body,
        out_shape=[
            jax.ShapeDtypeStruct((N, CLS), jnp.float32),
            jax.ShapeDtypeStruct((N, CLS), jnp.float32),
            jax.ShapeDtypeStruct((N, 1), jnp.float32),
        ],
    )(aggp, degp, xr, w1lt, w2lt, w2rt, b2)


def _tc_finish(agg2p, dinv, r):
    def body(agg2p_ref, dinv_ref, r_ref, o_ref):
        agg2 = jnp.concatenate([agg2p_ref[0, :N], agg2p_ref[1, :N]], axis=-1)
        m = agg2 * dinv_ref[...] + r_ref[...]
        mx = jnp.max(m, axis=1, keepdims=True)
        sh = m - mx
        o_ref[...] = sh - jnp.log(jnp.sum(jnp.exp(sh), axis=1, keepdims=True))

    return pl.pallas_call(
        body,
        out_shape=jax.ShapeDtypeStruct((N, CLS), jnp.float32),
    )(agg2p, dinv, r)


def kernel(x, edge_index, W1l, W1r, b1, W2l, W2r, b2):
    srcx = _tc_prep(edge_index[0].reshape(2500, 128)).reshape(
        NC, NS, CH2, C)
    dst16 = edge_index[1].reshape(NS, CH2, C)
    xf = x.reshape(2 * N, F_IN // 2)
    aggp, degp = _sc_pass1(xf, srcx, dst16)
    xr = _tc_dense1a(x, W1r.T, b1.reshape(1, HID))
    g, r, dinv = _tc_dense1b(aggp, degp, xr, W1l.T, W2l.T, W2r.T,
                             b2.reshape(1, CLS))
    gf = g.reshape(2 * N, CLS // 2)
    agg2p = _sc_pass2(gf, srcx, dst16)
    return _tc_finish(agg2p, dinv, r)
